# baseline (speedup 1.0000x reference)
import functools

import jax
import jax.numpy as jnp
from jax import lax
from jax.experimental import pallas as pl
from jax.experimental.pallas import tpu as pltpu
from jax.experimental.pallas import tpu_sc as plsc

_PAD = 0
_V = 32000
_B, _K, _L, _LS = 32, 16, 128, 128
_LANES = 16
_NSUB = 16


def _rerank_body(cand_hbm, batch_hbm, out_hbm, len_hbm, score_hbm,
                 cand_v, batch_v, reft, candt_a, candt_b, row_v, len16_v,
                 score_sv, len_shared, lens_v, dma_sem):
    c = lax.axis_index("c")
    s = lax.axis_index("s")
    b = c * _NSUB + s

    cand_dma = pltpu.async_copy(cand_hbm.at[b], cand_v, dma_sem)
    pltpu.sync_copy(batch_hbm.at[b], batch_v)

    zeros = jnp.zeros((_LANES,), jnp.float32)
    ones = jnp.ones((_LANES,), jnp.float32)
    lane = lax.iota(jnp.int32, _LANES)

    def bz_step(j, carry):
        t = batch_v[pl.ds(j * _LANES, _LANES)]
        plsc.store_scatter(reft, [t], zeros)
        return carry

    lax.fori_loop(0, _LS // _LANES, bz_step, 0)

    cand_dma.wait()

    def z_step(i, carry):
        t = cand_v[i >> 3, pl.ds((i & 7) * _LANES, _LANES)]
        plsc.store_scatter(reft, [t], zeros)
        return carry

    lax.fori_loop(0, _K * _L // _LANES, z_step, 0, unroll=4)

    def bh_step(j, carry):
        t = batch_v[pl.ds(j * _LANES, _LANES)]
        plsc.addupdate_scatter(reft, [t], ones, mask=t != _PAD)
        return carry

    lax.fori_loop(0, _LS // _LANES, bh_step, 0)

    def br_step(j, acc):
        t = batch_v[pl.ds(j * _LANES, _LANES)]
        return acc + plsc.load_gather(reft, [t])

    r2 = jnp.sum(lax.fori_loop(0, _LS // _LANES, br_step, zeros))

    def k_step(i, carry):
        c2s, dots = carry
        ka = 2 * i
        kb = 2 * i + 1

        def p1(j, acc):
            da, db = acc
            ta = cand_v[ka, pl.ds(j * _LANES, _LANES)]
            tb = cand_v[kb, pl.ds(j * _LANES, _LANES)]
            da = da + plsc.load_gather(reft, [ta])
            plsc.store_scatter(candt_a, [ta], zeros)
            db = db + plsc.load_gather(reft, [tb])
            plsc.store_scatter(candt_b, [tb], zeros)
            return da, db

        dota, dotb = lax.fori_loop(0, _L // _LANES, p1, (zeros, zeros))

        def p2(j, carry2):
            ta = cand_v[ka, pl.ds(j * _LANES, _LANES)]
            tb = cand_v[kb, pl.ds(j * _LANES, _LANES)]
            plsc.addupdate_scatter(candt_a, [ta], ones, mask=ta != _PAD)
            plsc.addupdate_scatter(candt_b, [tb], ones, mask=tb != _PAD)
            return carry2

        lax.fori_loop(0, _L // _LANES, p2, 0)

        def p3(j, acc):
            ca, cb = acc
            ta = cand_v[ka, pl.ds(j * _LANES, _LANES)]
            tb = cand_v[kb, pl.ds(j * _LANES, _LANES)]
            ca = ca + plsc.load_gather(candt_a, [ta])
            cb = cb + plsc.load_gather(candt_b, [tb])
            return ca, cb

        c2a, c2b = lax.fori_loop(0, _L // _LANES, p3, (zeros, zeros))

        c2s = jnp.where(lane == ka, jnp.broadcast_to(jnp.sum(c2a), (_LANES,)), c2s)
        c2s = jnp.where(lane == kb, jnp.broadcast_to(jnp.sum(c2b), (_LANES,)), c2s)
        dots = jnp.where(lane == ka, jnp.broadcast_to(jnp.sum(dota), (_LANES,)), dots)
        dots = jnp.where(lane == kb, jnp.broadcast_to(jnp.sum(dotb), (_LANES,)), dots)
        return c2s, dots

    c2s, dots = lax.fori_loop(0, _K // 2, k_step, (zeros, zeros))

    p = c2s * r2
    yi = jnp.int32(0x5F3759DF) - (plsc.bitcast(p, jnp.int32) >> 1)
    y = plsc.bitcast(yi, jnp.float32)
    for _ in range(3):
        y = y * (1.5 - 0.5 * p * y * y)
    scores = dots * y

    sorted_scores = jnp.sort(scores)

    m = jnp.min(scores)
    win = jnp.min(jnp.where(scores == m, lane, jnp.broadcast_to(jnp.int32(_K), (_LANES,))))

    winv = jnp.broadcast_to(win, (_LANES,))

    def w_step(j, nzacc):
        v = plsc.load_gather(cand_v, [winv, j * _LANES + lane])
        row_v[pl.ds(j * _LANES, _LANES)] = v
        return nzacc + (v != _PAD).astype(jnp.int32)

    nz = jnp.sum(lax.fori_loop(0, _L // _LANES, w_step, jnp.zeros((_LANES,), jnp.int32)))

    score_sv[...] = sorted_scores
    pltpu.sync_copy(row_v, out_hbm.at[b])
    pltpu.sync_copy(score_sv, score_hbm.at[b])

    len16_v[...] = jnp.broadcast_to(nz, (_LANES,))
    pltpu.sync_copy(len16_v, len_shared.at[pl.ds(s * _LANES, _LANES)])
    plsc.subcore_barrier()

    @pl.when(s == 0)
    def _():
        pltpu.sync_copy(len_shared, lens_v)
        diag = plsc.load_gather(lens_v, [lane * _LANES])
        len16_v[...] = diag
        pltpu.sync_copy(len16_v, len_hbm.at[pl.ds(c * _NSUB, _NSUB)])


_rerank = functools.partial(
    pl.kernel,
    out_type=(
        jax.ShapeDtypeStruct((_B, _L), jnp.int32),
        jax.ShapeDtypeStruct((_B,), jnp.int32),
        jax.ShapeDtypeStruct((_B, _K), jnp.float32),
    ),
    mesh=plsc.VectorSubcoreMesh(
        core_axis_name="c", subcore_axis_name="s", num_cores=2, num_subcores=16
    ),
    compiler_params=pltpu.CompilerParams(
        needs_layout_passes=False,
        skip_device_barrier=True,
        disable_bounds_checks=True,
        disable_semaphore_checks=True,
    ),
    scratch_types=(
        pltpu.VMEM((_K, _L), jnp.int32),
        pltpu.VMEM((_LS,), jnp.int32),
        pltpu.VMEM((_V,), jnp.float32),
        pltpu.VMEM((_V,), jnp.float32),
        pltpu.VMEM((_V,), jnp.float32),
        pltpu.VMEM((_L,), jnp.int32),
        pltpu.VMEM((_LANES,), jnp.int32),
        pltpu.VMEM((_LANES,), jnp.float32),
        pltpu.VMEM_SHARED((_NSUB * _LANES,), jnp.int32),
        pltpu.VMEM((_NSUB * _LANES,), jnp.int32),
        pltpu.SemaphoreType.DMA,
    ),
)(_rerank_body)


@jax.jit
def kernel(candidates, lengths, batch, tgt_field):
    del lengths, tgt_field
    out, lens, scores = _rerank(candidates, batch)
    return out, lens, scores


# speedup vs baseline: 1.0272x; 1.0272x over previous
import functools

import jax
import jax.numpy as jnp
from jax import lax
from jax.experimental import pallas as pl
from jax.experimental.pallas import tpu as pltpu
from jax.experimental.pallas import tpu_sc as plsc

_PAD = 0
_V = 32000
_B, _K, _L, _LS = 32, 16, 128, 128
_LANES = 16
_NSUB = 16


def _rerank_body(cand_hbm, batch_hbm, out_hbm, len_hbm, score_hbm,
                 cand_v, batch_v, reft, candt_a, candt_b, row_v, len16_v,
                 score_sv, len_shared, lens_v, dma_sem):
    c = lax.axis_index("c")
    s = lax.axis_index("s")
    b = c * _NSUB + s

    cand_dma = pltpu.async_copy(cand_hbm.at[b], cand_v, dma_sem)
    pltpu.sync_copy(batch_hbm.at[b], batch_v)

    zeros = jnp.zeros((_LANES,), jnp.float32)
    ones = jnp.ones((_LANES,), jnp.float32)
    lane = lax.iota(jnp.int32, _LANES)

    bt = [batch_v[pl.ds(j * _LANES, _LANES)] for j in range(_LS // _LANES)]
    for t in bt:
        plsc.store_scatter(reft, [t], zeros)

    cand_dma.wait()

    def z_step(i, carry):
        t = cand_v[i >> 3, pl.ds((i & 7) * _LANES, _LANES)]
        plsc.store_scatter(reft, [t], zeros)
        return carry

    lax.fori_loop(0, _K * _L // _LANES, z_step, 0, unroll=4)

    for t in bt:
        plsc.addupdate_scatter(reft, [t], ones, mask=t != _PAD)
    r2acc = zeros
    for t in bt:
        r2acc = r2acc + plsc.load_gather(reft, [t])
    r2 = jnp.sum(r2acc)

    def k_step(i, carry):
        c2s, dots = carry
        ka = 2 * i
        kb = 2 * i + 1
        cta = [cand_v[ka, pl.ds(j * _LANES, _LANES)] for j in range(_L // _LANES)]
        ctb = [cand_v[kb, pl.ds(j * _LANES, _LANES)] for j in range(_L // _LANES)]
        dota = zeros
        dotb = zeros
        for ta, tb in zip(cta, ctb):
            dota = dota + plsc.load_gather(reft, [ta])
            plsc.store_scatter(candt_a, [ta], zeros)
            dotb = dotb + plsc.load_gather(reft, [tb])
            plsc.store_scatter(candt_b, [tb], zeros)
        for ta, tb in zip(cta, ctb):
            plsc.addupdate_scatter(candt_a, [ta], ones, mask=ta != _PAD)
            plsc.addupdate_scatter(candt_b, [tb], ones, mask=tb != _PAD)
        c2a = zeros
        c2b = zeros
        for ta, tb in zip(cta, ctb):
            c2a = c2a + plsc.load_gather(candt_a, [ta])
            c2b = c2b + plsc.load_gather(candt_b, [tb])
        c2s = jnp.where(lane == ka, jnp.broadcast_to(jnp.sum(c2a), (_LANES,)), c2s)
        c2s = jnp.where(lane == kb, jnp.broadcast_to(jnp.sum(c2b), (_LANES,)), c2s)
        dots = jnp.where(lane == ka, jnp.broadcast_to(jnp.sum(dota), (_LANES,)), dots)
        dots = jnp.where(lane == kb, jnp.broadcast_to(jnp.sum(dotb), (_LANES,)), dots)
        return c2s, dots

    c2s, dots = lax.fori_loop(0, _K // 2, k_step, (zeros, zeros))

    p = c2s * r2
    yi = jnp.int32(0x5F3759DF) - (plsc.bitcast(p, jnp.int32) >> 1)
    y = plsc.bitcast(yi, jnp.float32)
    for _ in range(4):
        y = y * (1.5 - 0.5 * p * y * y)
    scores = dots * y

    sorted_scores = jnp.sort(scores)

    m = jnp.min(scores)
    win = jnp.min(jnp.where(scores == m, lane, jnp.broadcast_to(jnp.int32(_K), (_LANES,))))

    nz = jnp.int32(0)
    winv = jnp.broadcast_to(win, (_LANES,))
    for j in range(_L // _LANES):
        v = plsc.load_gather(cand_v, [winv, j * _LANES + lane])
        row_v[pl.ds(j * _LANES, _LANES)] = v
        nz = nz + jnp.sum((v != _PAD).astype(jnp.int32))

    score_sv[...] = sorted_scores
    row_dma = pltpu.async_copy(row_v, out_hbm.at[b], dma_sem)
    score_dma = pltpu.async_copy(score_sv, score_hbm.at[b], dma_sem)

    len16_v[...] = jnp.broadcast_to(nz, (_LANES,))
    pltpu.sync_copy(len16_v, len_shared.at[pl.ds(s * _LANES, _LANES)])
    plsc.subcore_barrier()

    @pl.when(s == 0)
    def _():
        pltpu.sync_copy(len_shared, lens_v)
        diag = plsc.load_gather(lens_v, [lane * _LANES])
        len16_v[...] = diag
        pltpu.sync_copy(len16_v, len_hbm.at[pl.ds(c * _NSUB, _NSUB)])

    row_dma.wait()
    score_dma.wait()


_rerank = functools.partial(
    pl.kernel,
    out_type=(
        jax.ShapeDtypeStruct((_B, _L), jnp.int32),
        jax.ShapeDtypeStruct((_B,), jnp.int32),
        jax.ShapeDtypeStruct((_B, _K), jnp.float32),
    ),
    mesh=plsc.VectorSubcoreMesh(
        core_axis_name="c", subcore_axis_name="s", num_cores=2, num_subcores=16
    ),
    compiler_params=pltpu.CompilerParams(
        needs_layout_passes=False,
        skip_device_barrier=True,
        disable_bounds_checks=True,
        disable_semaphore_checks=True,
    ),
    scratch_types=(
        pltpu.VMEM((_K, _L), jnp.int32),
        pltpu.VMEM((_LS,), jnp.int32),
        pltpu.VMEM((_V,), jnp.float32),
        pltpu.VMEM((_V,), jnp.float32),
        pltpu.VMEM((_V,), jnp.float32),
        pltpu.VMEM((_L,), jnp.int32),
        pltpu.VMEM((_LANES,), jnp.int32),
        pltpu.VMEM((_LANES,), jnp.float32),
        pltpu.VMEM_SHARED((_NSUB * _LANES,), jnp.int32),
        pltpu.VMEM((_NSUB * _LANES,), jnp.int32),
        pltpu.SemaphoreType.DMA,
    ),
)(_rerank_body)


@jax.jit
def kernel(candidates, lengths, batch, tgt_field):
    del lengths, tgt_field
    out, lens, scores = _rerank(candidates, batch)
    return out, lens, scores


# speedup vs baseline: 1.0296x; 1.0023x over previous
import functools

import jax
import jax.numpy as jnp
from jax import lax
from jax.experimental import pallas as pl
from jax.experimental.pallas import tpu as pltpu
from jax.experimental.pallas import tpu_sc as plsc

_PAD = 0
_V = 32000
_B, _K, _L, _LS = 32, 16, 128, 128
_LANES = 16
_NSUB = 16


def _rerank_body(cand_hbm, batch_hbm, out_hbm, len_hbm, score_hbm,
                 cand_v, batch_v, reft, candt_a, candt_b, row_v, len16_v,
                 score_sv, len_shared, lens_v, dma_sem):
    c = lax.axis_index("c")
    s = lax.axis_index("s")
    b = c * _NSUB + s

    cand_dma = pltpu.async_copy(cand_hbm.at[b], cand_v, dma_sem)
    pltpu.sync_copy(batch_hbm.at[b], batch_v)

    zeros = jnp.zeros((_LANES,), jnp.float32)
    ones = jnp.ones((_LANES,), jnp.float32)
    lane = lax.iota(jnp.int32, _LANES)

    bt = [batch_v[pl.ds(j * _LANES, _LANES)] for j in range(_LS // _LANES)]
    for t in bt:
        plsc.store_scatter(reft, [t], zeros)

    cand_dma.wait()

    def z_step(i, carry):
        t = cand_v[i >> 3, pl.ds((i & 7) * _LANES, _LANES)]
        plsc.store_scatter(reft, [t], zeros)
        return carry

    lax.fori_loop(0, _K * _L // _LANES, z_step, 0, unroll=4)

    for t in bt:
        plsc.addupdate_scatter(reft, [t], ones, mask=t != _PAD)
    r2acc = zeros
    for t in bt:
        r2acc = r2acc + plsc.load_gather(reft, [t])
    r2 = jnp.sum(r2acc)

    def k_step(i, carry):
        c2s, dots = carry
        ka = 2 * i
        kb = 2 * i + 1
        cta = [cand_v[ka, pl.ds(j * _LANES, _LANES)] for j in range(_L // _LANES)]
        ctb = [cand_v[kb, pl.ds(j * _LANES, _LANES)] for j in range(_L // _LANES)]
        dota = zeros
        dotb = zeros
        for ta, tb in zip(cta, ctb):
            dota = dota + plsc.load_gather(reft, [ta])
            plsc.store_scatter(candt_a, [ta], zeros)
            dotb = dotb + plsc.load_gather(reft, [tb])
            plsc.store_scatter(candt_b, [tb], zeros)
        for ta, tb in zip(cta, ctb):
            plsc.addupdate_scatter(candt_a, [ta], ones, mask=ta != _PAD)
            plsc.addupdate_scatter(candt_b, [tb], ones, mask=tb != _PAD)
        c2a = zeros
        c2b = zeros
        for ta, tb in zip(cta, ctb):
            c2a = c2a + plsc.load_gather(candt_a, [ta])
            c2b = c2b + plsc.load_gather(candt_b, [tb])
        c2s = jnp.where(lane == ka, jnp.broadcast_to(jnp.sum(c2a), (_LANES,)), c2s)
        c2s = jnp.where(lane == kb, jnp.broadcast_to(jnp.sum(c2b), (_LANES,)), c2s)
        dots = jnp.where(lane == ka, jnp.broadcast_to(jnp.sum(dota), (_LANES,)), dots)
        dots = jnp.where(lane == kb, jnp.broadcast_to(jnp.sum(dotb), (_LANES,)), dots)
        return c2s, dots

    c2s, dots = lax.fori_loop(0, _K // 2, k_step, (zeros, zeros))

    p = c2s * r2
    yi = jnp.int32(0x5F3759DF) - (plsc.bitcast(p, jnp.int32) >> 1)
    y = plsc.bitcast(yi, jnp.float32)
    for _ in range(4):
        y = y * (1.5 - 0.5 * p * y * y)
    scores = dots * y

    sorted_scores = jnp.sort(scores)

    m = jnp.min(scores)
    win = jnp.min(jnp.where(scores == m, lane, jnp.broadcast_to(jnp.int32(_K), (_LANES,))))

    nz = jnp.int32(0)
    winv = jnp.broadcast_to(win, (_LANES,))
    for j in range(_L // _LANES):
        v = plsc.load_gather(cand_v, [winv, j * _LANES + lane])
        row_v[pl.ds(j * _LANES, _LANES)] = v
        nz = nz + jnp.sum((v != _PAD).astype(jnp.int32))

    score_sv[...] = sorted_scores
    row_dma = pltpu.async_copy(row_v, out_hbm.at[b], dma_sem)
    score_dma = pltpu.async_copy(score_sv, score_hbm.at[b], dma_sem)

    len16_v[...] = jnp.broadcast_to(nz, (_LANES,))
    pltpu.sync_copy(len16_v, len_shared.at[pl.ds(s * _LANES, _LANES)])
    plsc.subcore_barrier()

    @pl.when(s == 0)
    def _():
        pltpu.sync_copy(len_shared, lens_v)
        diag = plsc.load_gather(lens_v, [lane * _LANES])
        len16_v[...] = diag
        pltpu.sync_copy(len16_v, len_hbm.at[pl.ds(c * _NSUB, _NSUB)])

    row_dma.wait()
    score_dma.wait()


@functools.cache
def _build_rerank():
    return functools.partial(
        pl.kernel,
        out_type=(
            jax.ShapeDtypeStruct((_B, _L), jnp.int32),
            jax.ShapeDtypeStruct((_B,), jnp.int32),
            jax.ShapeDtypeStruct((_B, _K), jnp.float32),
        ),
        mesh=plsc.VectorSubcoreMesh(
            core_axis_name="c", subcore_axis_name="s", num_cores=2, num_subcores=16
        ),
        compiler_params=pltpu.CompilerParams(
            needs_layout_passes=False,
            skip_device_barrier=True,
            disable_bounds_checks=True,
            disable_semaphore_checks=True,
        ),
        scratch_types=(
            pltpu.VMEM((_K, _L), jnp.int32),
            pltpu.VMEM((_LS,), jnp.int32),
            pltpu.VMEM((_V,), jnp.float32),
            pltpu.VMEM((_V,), jnp.float32),
            pltpu.VMEM((_V,), jnp.float32),
            pltpu.VMEM((_L,), jnp.int32),
            pltpu.VMEM((_LANES,), jnp.int32),
            pltpu.VMEM((_LANES,), jnp.float32),
            pltpu.VMEM_SHARED((_NSUB * _LANES,), jnp.int32),
            pltpu.VMEM((_NSUB * _LANES,), jnp.int32),
            pltpu.SemaphoreType.DMA,
        ),
    )(_rerank_body)


@jax.jit
def kernel(candidates, lengths, batch, tgt_field):
    del lengths, tgt_field
    out, lens, scores = _build_rerank()(candidates, batch)
    return out, lens, scores
